# Initial kernel scaffold; baseline (speedup 1.0000x reference)
import jax
import jax.numpy as jnp
from jax.experimental import pallas as pl


def kernel(cls_pred, point_coord_pred, point_confidence_pred, matched_src_idx, matched_gt_idx, gt_class, gt_points, gt_pt_padding_flags, gt_num):
    raise NotImplementedError("write your pallas kernel here")


# speedup vs baseline: 1.5223x; 1.5223x over previous
import functools

import jax
import jax.numpy as jnp
from jax import lax
from jax.experimental import pallas as pl
from jax.experimental.pallas import tpu as pltpu
from jax.experimental.pallas import tpu_sc as plsc

NUM_CLASSES = 5
BACKGROUND = 4
GAMMA = 2.0
ALPHA_BG = 0.25
CLASS_W = 2.0
PT_CONF_W = 1.0
PT_COORD_W = 5.0
PAD_VALUE = -10000.0

B, Q, G, P = 8, 512, 128, 64
N_PAIR = B * G
N_ROW = B * Q
CPAD = 8


def _sc_assemble(src_flat, gt_flat, conf_tab, coord_tab, gtpt_tab, flags_tab,
                 cls_tab):
    info = plsc.get_sparse_core_info()
    nc, ns = info.num_cores, info.num_subcores
    nw = nc * ns
    ppw = N_PAIR // nw
    mesh = plsc.VectorSubcoreMesh(core_axis_name="c", subcore_axis_name="s")

    @functools.partial(
        pl.kernel,
        mesh=mesh,
        compiler_params=pltpu.CompilerParams(use_tc_tiling_on_sc=False),
        out_type=[
            jax.ShapeDtypeStruct((N_PAIR, P), jnp.float32),
            jax.ShapeDtypeStruct((N_PAIR, CPAD), jnp.float32),
            jax.ShapeDtypeStruct((nw, 16), jnp.float32),
            jax.ShapeDtypeStruct((nw, 16), jnp.float32),
        ],
        scratch_types=[
            pltpu.VMEM((ppw,), jnp.int32),
            pltpu.VMEM((ppw,), jnp.int32),
            pltpu.VMEM((ppw, P), jnp.float32),
            pltpu.VMEM((ppw, P), jnp.int32),
            pltpu.VMEM((ppw, 2 * P), jnp.float32),
            pltpu.VMEM((ppw, 2 * P), jnp.float32),
            pltpu.VMEM((ppw, CPAD), jnp.float32),
            pltpu.VMEM((ppw, P), jnp.float32),
            pltpu.VMEM((16,), jnp.float32),
            pltpu.VMEM((16,), jnp.float32),
            pltpu.SemaphoreType.DMA,
        ],
    )
    def sc_kernel(src_hbm, gt_hbm, conf_hbm, coord_hbm, gtpt_hbm, flags_hbm,
                  cls_hbm, q_out, mcls_out, l1_out, cnt_out,
                  idx_v, gtv, conf_v, flag_v, coord_v, gtpt_v, cls_v, q_v,
                  l1_v, cnt_v, sem):
        wid = lax.axis_index("s") * nc + lax.axis_index("c")
        base = wid * ppw

        pltpu.sync_copy(src_hbm.at[pl.ds(base, ppw)], idx_v)
        pltpu.sync_copy(gt_hbm.at[pl.ds(base, ppw)], gtv)

        pltpu.async_copy(conf_hbm.at[idx_v], conf_v, sem).wait()
        pltpu.async_copy(flags_hbm.at[gtv], flag_v, sem).wait()
        pltpu.async_copy(coord_hbm.at[idx_v], coord_v, sem).wait()
        pltpu.async_copy(gtpt_hbm.at[gtv], gtpt_v, sem).wait()
        pltpu.async_copy(cls_hbm.at[idx_v], cls_v, sem).wait()

        pltpu.sync_copy(cls_v, mcls_out.at[pl.ds(base, ppw)])

        def qrow(r, _):
            for c in range(P // 16):
                sl = pl.ds(c * 16, 16)
                t = flag_v[r, sl]
                p = conf_v[r, sl]
                q_v[r, sl] = jnp.where(t != 0, p, 1.0 - p)
            return 0

        lax.fori_loop(0, ppw, qrow, 0)
        pltpu.sync_copy(q_v, q_out.at[pl.ds(base, ppw)])

        def crow(r, carry):
            l1, cnt = carry
            for c in range(2 * P // 16):
                sl = pl.ds(c * 16, 16)
                tp = gtpt_v[r, sl]
                sp = coord_v[r, sl]
                m = jnp.where(tp != PAD_VALUE, 1.0, 0.0)
                l1 = l1 + jnp.abs(sp - tp) * m
                cnt = cnt + m
            return (l1, cnt)

        zero = jnp.zeros((16,), jnp.float32)
        l1, cnt = lax.fori_loop(0, ppw, crow, (zero, zero))
        l1_v[...] = l1
        cnt_v[...] = cnt
        pltpu.sync_copy(l1_v, l1_out.at[wid])
        pltpu.sync_copy(cnt_v, cnt_out.at[wid])

    return sc_kernel(src_flat, gt_flat, conf_tab, coord_tab, gtpt_tab,
                     flags_tab, cls_tab)


def _tc_body(cls_ref, src_c_ref, src_r_ref, gt_c_ref, gcls_ref, mcls_ref,
             q_ref, l1_ref, cnt_ref, out_ref):
    f32 = jnp.float32

    def masked_logits(x):
        lane = lax.broadcasted_iota(jnp.int32, x.shape, 1)
        return jnp.where(lane < NUM_CLASSES, x, -1e30)

    def lse(x):
        m = jnp.max(x, axis=1, keepdims=True)
        return jnp.log(jnp.sum(jnp.exp(x - m), axis=1, keepdims=True)) + m

    x = masked_logits(cls_ref[...])
    ls4 = x[:, BACKGROUND:BACKGROUND + 1] - lse(x)
    p4 = jnp.exp(ls4)
    base_rows = -(1.0 - ALPHA_BG) * (1.0 - p4) * (1.0 - p4) * ls4
    base_sum = jnp.sum(base_rows)

    corr = f32(0.0)
    for b in range(B):
        rs = pl.ds(b * G, G)
        sc = src_c_ref[rs, :]
        sr = src_r_ref[:, rs]
        eqm = sc == sr
        gi = lax.broadcasted_iota(jnp.int32, (G, G), 0)
        gj = lax.broadcasted_iota(jnp.int32, (G, G), 1)
        later = jnp.where(eqm & (gj > gi), 1, 0)
        conflict = jnp.max(later, axis=1, keepdims=True)
        winner = (conflict == 0).astype(f32)

        gc = gt_c_ref[rs, :]
        gmat = gc == (gj + b * G)
        gclsr = gcls_ref[:, rs].astype(f32)
        tcls = jnp.sum(jnp.where(gmat, gclsr, 0.0), axis=1, keepdims=True)

        xm = masked_logits(mcls_ref[rs, :])
        lsoft = xm - lse(xm)
        lane = lax.broadcasted_iota(jnp.int32, (G, CPAD), 1).astype(f32)
        onehot = (lane == tcls).astype(f32)
        logp_t = jnp.sum(lsoft * onehot, axis=1, keepdims=True)
        p_t = jnp.exp(logp_t)
        alpha = jnp.where(tcls == 0.0, ALPHA_BG, 1.0 - ALPHA_BG)
        loss_new = -alpha * (1.0 - p_t) * (1.0 - p_t) * logp_t
        ls4m = lsoft[:, BACKGROUND:BACKGROUND + 1]
        p4m = jnp.exp(ls4m)
        loss_old = -(1.0 - ALPHA_BG) * (1.0 - p4m) * (1.0 - p4m) * ls4m
        corr = corr + jnp.sum(winner * (loss_new - loss_old))

    class_loss = CLASS_W * (base_sum + corr) / f32(N_ROW)

    conf_loss = PT_CONF_W * (-jnp.sum(jnp.log(q_ref[...])) / f32(N_PAIR * P))

    l1s = jnp.sum(l1_ref[...])
    cnts = jnp.sum(cnt_ref[...])
    coord_loss = PT_COORD_W * l1s / jnp.maximum(cnts, 1.0)

    lane = lax.broadcasted_iota(jnp.int32, (1, 128), 1)
    out = (jnp.where(lane == 0, class_loss, 0.0)
           + jnp.where(lane == 1, conf_loss, 0.0)
           + jnp.where(lane == 2, coord_loss, 0.0))
    out_ref[...] = out.astype(f32)


def _tc_losses(cls_tab, src_col, src_row, gt_col, gcls_row, mcls, q2d,
               l1_part, cnt_part):
    return pl.pallas_call(
        _tc_body,
        out_shape=jax.ShapeDtypeStruct((1, 128), jnp.float32),
    )(cls_tab, src_col, src_row, gt_col, gcls_row, mcls, q2d, l1_part,
      cnt_part)


def kernel(cls_pred, point_coord_pred, point_confidence_pred,
           matched_src_idx, matched_gt_idx, gt_class, gt_points,
           gt_pt_padding_flags, gt_num):
    i32 = jnp.int32
    src = matched_src_idx.astype(i32)
    mgt = matched_gt_idx.astype(i32)
    boff_q = (jnp.arange(B, dtype=i32) * Q)[:, None]
    boff_g = (jnp.arange(B, dtype=i32) * G)[:, None]
    src_flat = (boff_q + src).reshape(-1)
    gt_flat = (boff_g + mgt).reshape(-1)

    conf_tab = point_confidence_pred.reshape(N_ROW, P)
    coord_tab = point_coord_pred.reshape(N_ROW, 2 * P)
    gtpt_tab = gt_points.reshape(N_PAIR, 2 * P)
    flags_tab = gt_pt_padding_flags.astype(i32).reshape(N_PAIR, P)
    cls_flat = cls_pred.reshape(N_ROW, NUM_CLASSES)
    cls_tab = jnp.concatenate(
        [cls_flat, jnp.zeros((N_ROW, CPAD - NUM_CLASSES), jnp.float32)],
        axis=1)

    q2d, mcls, l1_part, cnt_part = _sc_assemble(
        src_flat, gt_flat, conf_tab, coord_tab, gtpt_tab, flags_tab, cls_tab)

    out = _tc_losses(cls_tab, src_flat.reshape(N_PAIR, 1),
                     src_flat.reshape(1, N_PAIR), gt_flat.reshape(N_PAIR, 1),
                     gt_class.astype(i32).reshape(1, N_PAIR), mcls, q2d,
                     l1_part, cnt_part)
    return (out[0, 0], out[0, 1], out[0, 2])


# speedup vs baseline: 2.1647x; 1.4220x over previous
import functools

import jax
import jax.numpy as jnp
from jax import lax
from jax.experimental import pallas as pl
from jax.experimental.pallas import tpu as pltpu
from jax.experimental.pallas import tpu_sc as plsc

NUM_CLASSES = 5
BACKGROUND = 4
GAMMA = 2.0
ALPHA_BG = 0.25
CLASS_W = 2.0
PT_CONF_W = 1.0
PT_COORD_W = 5.0
PAD_VALUE = -10000.0

B, Q, G, P = 8, 512, 128, 64
N_PAIR = B * G
N_ROW = B * Q


def _sc_assemble(srcm, mgtm, conf_t, coord_t, gtpt_t, flags_t, cls_t):
    info = plsc.get_sparse_core_info()
    nc, ns = info.num_cores, info.num_subcores
    nw = nc * ns
    pts = P // (nw // B)
    mesh = plsc.VectorSubcoreMesh(core_axis_name="c", subcore_axis_name="s")

    @functools.partial(
        pl.kernel,
        mesh=mesh,
        compiler_params=pltpu.CompilerParams(needs_layout_passes=False),
        out_type=[
            jax.ShapeDtypeStruct((P, N_PAIR), jnp.float32),
            jax.ShapeDtypeStruct((NUM_CLASSES * N_PAIR,), jnp.float32),
            jax.ShapeDtypeStruct((nw * 16,), jnp.float32),
            jax.ShapeDtypeStruct((nw * 16,), jnp.float32),
        ],
        scratch_types=[
            pltpu.VMEM((B, G), jnp.int32),
            pltpu.VMEM((B, G), jnp.int32),
            pltpu.VMEM((pts, Q), jnp.float32),
            pltpu.VMEM((pts, G), jnp.int32),
            pltpu.VMEM((pts, 2, Q), jnp.float32),
            pltpu.VMEM((pts, 2, G), jnp.float32),
            pltpu.VMEM((B, Q), jnp.float32),
            pltpu.VMEM((B, Q), jnp.float32),
            pltpu.VMEM((pts, G), jnp.float32),
            pltpu.VMEM((2, G), jnp.float32),
            pltpu.VMEM((16,), jnp.float32),
            pltpu.VMEM((16,), jnp.float32),
            pltpu.SemaphoreType.DMA,
            pltpu.SemaphoreType.DMA,
        ],
    )
    def sc_kernel(src_hbm, mgt_hbm, conf_hbm, coord_hbm, gtpt_hbm, flags_hbm,
                  cls_hbm, q_out, mcls_out, l1_out, cnt_out,
                  src_all, mgt_all, conf_sl, flag_sl, coord_sl, gtpt_sl,
                  cls_sl, cls_sl2, q_sl, mcls_sl, l1_v, cnt_v, sem, osem):
        wid = lax.axis_index("s") * nc + lax.axis_index("c")
        b = wid // (nw // B)
        p0 = (wid % (nw // B)) * pts
        cslab = wid // B
        cb = wid % B

        pend = [
            pltpu.async_copy(src_hbm, src_all, sem),
            pltpu.async_copy(mgt_hbm, mgt_all, sem),
            pltpu.async_copy(conf_hbm.at[b, pl.ds(p0, pts)], conf_sl, sem),
            pltpu.async_copy(coord_hbm.at[b, pl.ds(p0, pts)], coord_sl, sem),
            pltpu.async_copy(
                flags_hbm.at[pl.ds(p0, pts), pl.ds(b * G, G)], flag_sl, sem),
            pltpu.async_copy(
                gtpt_hbm.at[pl.ds(p0, pts), :, pl.ds(b * G, G)], gtpt_sl,
                sem),
            pltpu.async_copy(cls_hbm.at[cslab], cls_sl, sem),
        ]

        @pl.when(wid < B)
        def _():
            pltpu.async_copy(cls_hbm.at[NUM_CLASSES - 1], cls_sl2, sem).wait()
        for cp in pend:
            cp.wait()

        opend = []
        for i in range(pts):
            for k in range(G // 16):
                sl = pl.ds(k * 16, 16)
                row = jnp.full((16,), i, jnp.int32)
                idx = src_all[b, sl]
                gti = mgt_all[b, sl]
                pv = plsc.load_gather(conf_sl, [row, idx])
                fv = plsc.load_gather(flag_sl, [row, gti])
                q_sl[i, sl] = jnp.where(fv != 0, pv, 1.0 - pv)
        opend.append(pltpu.async_copy(
            q_sl, q_out.at[pl.ds(p0, pts), pl.ds(b * G, G)], osem))

        cbv = jnp.full((16,), cb, jnp.int32)
        for k in range(G // 16):
            sl = pl.ds(k * 16, 16)
            idx = src_all[cb, sl]
            mcls_sl[0, sl] = plsc.load_gather(cls_sl, [cbv, idx])
        opend.append(pltpu.async_copy(
            mcls_sl.at[0],
            mcls_out.at[pl.ds(cslab * N_PAIR + cb * G, G)], osem))

        @pl.when(wid < B)
        def _():
            for k in range(G // 16):
                sl = pl.ds(k * 16, 16)
                idx = src_all[cb, sl]
                mcls_sl[1, sl] = plsc.load_gather(cls_sl2, [cbv, idx])
            pltpu.async_copy(
                mcls_sl.at[1],
                mcls_out.at[pl.ds((NUM_CLASSES - 1) * N_PAIR + cb * G, G)],
                osem).wait()

        l1 = jnp.zeros((16,), jnp.float32)
        cnt = jnp.zeros((16,), jnp.float32)
        for i in range(pts):
            for c in range(2):
                rowv = jnp.full((16,), i, jnp.int32)
                cv = jnp.full((16,), c, jnp.int32)
                for k in range(G // 16):
                    sl = pl.ds(k * 16, 16)
                    sp = plsc.load_gather(coord_sl,
                                          [rowv, cv, src_all[b, sl]])
                    tp = plsc.load_gather(gtpt_sl,
                                          [rowv, cv, mgt_all[b, sl]])
                    m = jnp.where(tp != PAD_VALUE, 1.0, 0.0)
                    l1 = l1 + jnp.abs(sp - tp) * m
                    cnt = cnt + m
        l1_v[...] = l1
        cnt_v[...] = cnt
        opend.append(pltpu.async_copy(l1_v, l1_out.at[pl.ds(wid * 16, 16)],
                                      osem))
        opend.append(pltpu.async_copy(cnt_v, cnt_out.at[pl.ds(wid * 16, 16)],
                                      osem))
        for cp in opend:
            cp.wait()

    return sc_kernel(srcm, mgtm, conf_t, coord_t, gtpt_t, flags_t, cls_t)


def _tc_body(cls_ref, src_c_ref, src_r_ref, mgt_r_ref, gcls_c_ref, mcls_ref,
             q_ref, l1_ref, cnt_ref, out_ref):
    f32 = jnp.float32

    def lse0(x):
        m = jnp.max(x, axis=0, keepdims=True)
        return jnp.log(jnp.sum(jnp.exp(x - m), axis=0, keepdims=True)) + m

    x = cls_ref[...]
    ls4 = x[BACKGROUND:BACKGROUND + 1, :] - lse0(x)
    p4 = jnp.exp(ls4)
    base_sum = jnp.sum(-(1.0 - ALPHA_BG) * (1.0 - p4) * (1.0 - p4) * ls4)

    corr = f32(0.0)
    for b in range(B):
        cs = pl.ds(b * G, G)
        sc = src_c_ref[cs, :]
        sr = src_r_ref[:, cs]
        gi = lax.broadcasted_iota(jnp.int32, (G, G), 0)
        gj = lax.broadcasted_iota(jnp.int32, (G, G), 1)
        later = jnp.where((sc == sr) & (gi > gj), 1, 0)
        conflict = jnp.max(later, axis=0, keepdims=True)
        winner = (conflict == 0).astype(f32)

        mr = mgt_r_ref[:, cs]
        gmat = mr == gi
        gcls = gcls_c_ref[cs, :].astype(f32)
        tcls = jnp.sum(jnp.where(gmat, gcls, 0.0), axis=0, keepdims=True)

        xm = mcls_ref[:, cs]
        lsoft = xm - lse0(xm)
        lane0 = lax.broadcasted_iota(jnp.int32, (NUM_CLASSES, G), 0)
        onehot = (lane0.astype(f32) == tcls).astype(f32)
        logp_t = jnp.sum(lsoft * onehot, axis=0, keepdims=True)
        p_t = jnp.exp(logp_t)
        alpha = jnp.where(tcls == 0.0, ALPHA_BG, 1.0 - ALPHA_BG)
        loss_new = -alpha * (1.0 - p_t) * (1.0 - p_t) * logp_t
        ls4m = lsoft[BACKGROUND:BACKGROUND + 1, :]
        p4m = jnp.exp(ls4m)
        loss_old = -(1.0 - ALPHA_BG) * (1.0 - p4m) * (1.0 - p4m) * ls4m
        corr = corr + jnp.sum(winner * (loss_new - loss_old))

    class_loss = CLASS_W * (base_sum + corr) / f32(N_ROW)

    conf_loss = PT_CONF_W * (-jnp.sum(jnp.log(q_ref[...])) / f32(N_PAIR * P))

    l1s = jnp.sum(l1_ref[...])
    cnts = jnp.sum(cnt_ref[...])
    coord_loss = PT_COORD_W * l1s / jnp.maximum(cnts, 1.0)

    lane = lax.broadcasted_iota(jnp.int32, (1, 128), 1)
    out = (jnp.where(lane == 0, class_loss, 0.0)
           + jnp.where(lane == 1, conf_loss, 0.0)
           + jnp.where(lane == 2, coord_loss, 0.0))
    out_ref[...] = out.astype(f32)


def _tc_losses(cls_t2, src_col, src_row, mgt_row, gcls_col, mcls, q,
               l1_part, cnt_part):
    return pl.pallas_call(
        _tc_body,
        out_shape=jax.ShapeDtypeStruct((1, 128), jnp.float32),
    )(cls_t2, src_col, src_row, mgt_row, gcls_col, mcls, q, l1_part, cnt_part)


def kernel(cls_pred, point_coord_pred, point_confidence_pred,
           matched_src_idx, matched_gt_idx, gt_class, gt_points,
           gt_pt_padding_flags, gt_num):
    i32 = jnp.int32
    srcm = matched_src_idx.astype(i32)
    mgtm = matched_gt_idx.astype(i32)

    conf_t = jnp.transpose(point_confidence_pred, (0, 2, 1))
    coord_t = jnp.transpose(point_coord_pred, (0, 2, 3, 1))
    gtpt_t = jnp.transpose(gt_points, (1, 2, 0))
    flags_t = jnp.transpose(gt_pt_padding_flags.astype(i32), (1, 0))
    cls_t = jnp.transpose(cls_pred, (2, 0, 1))

    q, mcls, l1_part, cnt_part = _sc_assemble(
        srcm, mgtm, conf_t, coord_t, gtpt_t, flags_t, cls_t)

    out = _tc_losses(cls_t.reshape(NUM_CLASSES, N_ROW),
                     srcm.reshape(N_PAIR, 1), srcm.reshape(1, N_PAIR),
                     mgtm.reshape(1, N_PAIR),
                     gt_class.astype(i32).reshape(N_PAIR, 1),
                     mcls.reshape(NUM_CLASSES, N_PAIR), q,
                     l1_part.reshape(4, 128), cnt_part.reshape(4, 128))
    return (out[0, 0], out[0, 1], out[0, 2])


# speedup vs baseline: 2.2603x; 1.0442x over previous
import functools

import jax
import jax.numpy as jnp
from jax import lax
from jax.experimental import pallas as pl
from jax.experimental.pallas import tpu as pltpu
from jax.experimental.pallas import tpu_sc as plsc

NUM_CLASSES = 5
BACKGROUND = 4
GAMMA = 2.0
ALPHA_BG = 0.25
CLASS_W = 2.0
PT_CONF_W = 1.0
PT_COORD_W = 5.0
PAD_VALUE = -10000.0

B, Q, G, P = 8, 512, 128, 64
N_PAIR = B * G
N_ROW = B * Q


def _sc_assemble(srcm, mgtm, conf_t, coord_t, gtpt_t, flags_t, cls_t):
    info = plsc.get_sparse_core_info()
    nc, ns = info.num_cores, info.num_subcores
    nw = nc * ns
    pts = P // (nw // B)
    mesh = plsc.VectorSubcoreMesh(core_axis_name="c", subcore_axis_name="s")

    @functools.partial(
        pl.kernel,
        mesh=mesh,
        compiler_params=pltpu.CompilerParams(needs_layout_passes=False),
        out_type=[
            jax.ShapeDtypeStruct((P, N_PAIR), jnp.float32),
            jax.ShapeDtypeStruct((NUM_CLASSES, N_PAIR), jnp.float32),
            jax.ShapeDtypeStruct((nw * 16,), jnp.float32),
            jax.ShapeDtypeStruct((nw * 16,), jnp.float32),
        ],
        scratch_types=[
            pltpu.VMEM((B, G), jnp.int32),
            pltpu.VMEM((B, G), jnp.int32),
            pltpu.VMEM((pts, Q), jnp.float32),
            pltpu.VMEM((pts, G), jnp.int32),
            pltpu.VMEM((pts, 2, Q), jnp.float32),
            pltpu.VMEM((pts, 2, G), jnp.float32),
            pltpu.VMEM((NUM_CLASSES, Q), jnp.float32),
            pltpu.VMEM((pts, G), jnp.float32),
            pltpu.VMEM((NUM_CLASSES, G), jnp.float32),
            pltpu.VMEM((16,), jnp.float32),
            pltpu.VMEM((16,), jnp.float32),
            pltpu.SemaphoreType.DMA,
            pltpu.SemaphoreType.DMA,
        ],
    )
    def sc_kernel(src_hbm, mgt_hbm, conf_hbm, coord_hbm, gtpt_hbm, flags_hbm,
                  cls_hbm, q_out, mcls_out, l1_out, cnt_out,
                  src_all, mgt_all, conf_sl, flag_sl, coord_sl, gtpt_sl,
                  cls_sl, q_sl, mcls_sl, l1_v, cnt_v, sem, osem):
        wid = lax.axis_index("s") * nc + lax.axis_index("c")
        b = wid // (nw // B)
        p0 = (wid % (nw // B)) * pts

        pend = [
            pltpu.async_copy(src_hbm, src_all, sem),
            pltpu.async_copy(mgt_hbm, mgt_all, sem),
            pltpu.async_copy(conf_hbm.at[b, pl.ds(p0, pts)], conf_sl, sem),
            pltpu.async_copy(coord_hbm.at[b, pl.ds(p0, pts)], coord_sl, sem),
            pltpu.async_copy(
                flags_hbm.at[pl.ds(p0, pts), pl.ds(b * G, G)], flag_sl, sem),
            pltpu.async_copy(
                gtpt_hbm.at[pl.ds(p0, pts), :, pl.ds(b * G, G)], gtpt_sl,
                sem),
        ]

        @pl.when(wid < B)
        def _():
            pltpu.async_copy(cls_hbm.at[:, wid % B], cls_sl, sem).wait()
        for cp in pend:
            cp.wait()

        opend = []
        l1 = jnp.zeros((16,), jnp.float32)
        cnt = jnp.zeros((16,), jnp.float32)
        for k in range(G // 16):
            sl = pl.ds(k * 16, 16)
            idx = src_all[b, sl]
            gti = mgt_all[b, sl]
            for i in range(pts):
                row = jnp.full((16,), i, jnp.int32)
                pv = plsc.load_gather(conf_sl, [row, idx])
                fv = plsc.load_gather(flag_sl, [row, gti])
                q_sl[i, sl] = jnp.where(fv != 0, pv, 1.0 - pv)
                for c in range(2):
                    cv = jnp.full((16,), c, jnp.int32)
                    sp = plsc.load_gather(coord_sl, [row, cv, idx])
                    tp = plsc.load_gather(gtpt_sl, [row, cv, gti])
                    m = jnp.where(tp != PAD_VALUE, 1.0, 0.0)
                    l1 = l1 + jnp.abs(sp - tp) * m
                    cnt = cnt + m
        opend.append(pltpu.async_copy(
            q_sl, q_out.at[pl.ds(p0, pts), pl.ds(b * G, G)], osem))

        @pl.when(wid < B)
        def _():
            cb = wid % B
            for k in range(G // 16):
                sl = pl.ds(k * 16, 16)
                idx = src_all[cb, sl]
                for cc in range(NUM_CLASSES):
                    mcls_sl[cc, sl] = plsc.load_gather(
                        cls_sl, [jnp.full((16,), cc, jnp.int32), idx])
            pltpu.async_copy(
                mcls_sl, mcls_out.at[:, pl.ds(cb * G, G)], osem).wait()

        l1_v[...] = l1
        cnt_v[...] = cnt
        opend.append(pltpu.async_copy(l1_v, l1_out.at[pl.ds(wid * 16, 16)],
                                      osem))
        opend.append(pltpu.async_copy(cnt_v, cnt_out.at[pl.ds(wid * 16, 16)],
                                      osem))
        for cp in opend:
            cp.wait()

    return sc_kernel(srcm, mgtm, conf_t, coord_t, gtpt_t, flags_t, cls_t)


def _tc_body(cls_ref, src_r_ref, mgt_r_ref, gcls_r_ref, mcls_ref,
             q_ref, l1_ref, cnt_ref, out_ref):
    f32 = jnp.float32

    def lse0(x):
        m = jnp.max(x, axis=0, keepdims=True)
        return jnp.log(jnp.sum(jnp.exp(x - m), axis=0, keepdims=True)) + m

    eye = (lax.broadcasted_iota(jnp.int32, (G, G), 0)
           == lax.broadcasted_iota(jnp.int32, (G, G), 1))

    def to_col(vrow):
        return jnp.sum(jnp.where(eye, vrow, 0), axis=1, keepdims=True)

    x = cls_ref[...]
    ls4 = x[BACKGROUND:BACKGROUND + 1] - lse0(x)
    p4 = jnp.exp(ls4)
    base_sum = jnp.sum(-(1.0 - ALPHA_BG) * (1.0 - p4) * (1.0 - p4) * ls4)

    corr = f32(0.0)
    for b in range(B):
        cs = pl.ds(b * G, G)
        sr = src_r_ref[:, cs]
        sc = to_col(sr)
        gi = lax.broadcasted_iota(jnp.int32, (G, G), 0)
        gj = lax.broadcasted_iota(jnp.int32, (G, G), 1)
        later = jnp.where((sc == sr) & (gi > gj), 1, 0)
        conflict = jnp.max(later, axis=0, keepdims=True)
        winner = (conflict == 0).astype(f32)

        mr = mgt_r_ref[:, cs]
        gmat = mr == gi
        gcls = to_col(gcls_r_ref[:, cs]).astype(f32)
        tcls = jnp.sum(jnp.where(gmat, gcls, 0.0), axis=0, keepdims=True)

        xm = mcls_ref[:, cs]
        lsoft = xm - lse0(xm)
        lane0 = lax.broadcasted_iota(jnp.int32, (NUM_CLASSES, G), 0)
        onehot = (lane0.astype(f32) == tcls).astype(f32)
        logp_t = jnp.sum(lsoft * onehot, axis=0, keepdims=True)
        p_t = jnp.exp(logp_t)
        alpha = jnp.where(tcls == 0.0, ALPHA_BG, 1.0 - ALPHA_BG)
        loss_new = -alpha * (1.0 - p_t) * (1.0 - p_t) * logp_t
        ls4m = lsoft[BACKGROUND:BACKGROUND + 1, :]
        p4m = jnp.exp(ls4m)
        loss_old = -(1.0 - ALPHA_BG) * (1.0 - p4m) * (1.0 - p4m) * ls4m
        corr = corr + jnp.sum(winner * (loss_new - loss_old))

    class_loss = CLASS_W * (base_sum + corr) / f32(N_ROW)

    conf_loss = PT_CONF_W * (-jnp.sum(jnp.log(q_ref[...])) / f32(N_PAIR * P))

    l1s = jnp.sum(l1_ref[...])
    cnts = jnp.sum(cnt_ref[...])
    coord_loss = PT_COORD_W * l1s / jnp.maximum(cnts, 1.0)

    lane = lax.broadcasted_iota(jnp.int32, (1, 128), 1)
    out = (jnp.where(lane == 0, class_loss, 0.0)
           + jnp.where(lane == 1, conf_loss, 0.0)
           + jnp.where(lane == 2, coord_loss, 0.0))
    out_ref[...] = out.astype(f32)


def _tc_losses(cls_t, src_row, mgt_row, gcls_row, mcls, q, l1_part, cnt_part):
    return pl.pallas_call(
        _tc_body,
        out_shape=jax.ShapeDtypeStruct((1, 128), jnp.float32),
    )(cls_t, src_row, mgt_row, gcls_row, mcls, q, l1_part, cnt_part)


def kernel(cls_pred, point_coord_pred, point_confidence_pred,
           matched_src_idx, matched_gt_idx, gt_class, gt_points,
           gt_pt_padding_flags, gt_num):
    i32 = jnp.int32
    srcm = matched_src_idx.astype(i32)
    mgtm = matched_gt_idx.astype(i32)

    conf_t = jnp.transpose(point_confidence_pred, (0, 2, 1))
    coord_t = jnp.transpose(point_coord_pred, (0, 2, 3, 1))
    gtpt_t = jnp.transpose(gt_points, (1, 2, 0))
    flags_t = jnp.transpose(gt_pt_padding_flags.astype(i32), (1, 0))
    cls_t = jnp.transpose(cls_pred, (2, 0, 1))

    q, mcls, l1_part, cnt_part = _sc_assemble(
        srcm, mgtm, conf_t, coord_t, gtpt_t, flags_t, cls_t)

    out = _tc_losses(cls_t, srcm.reshape(1, N_PAIR), mgtm.reshape(1, N_PAIR),
                     gt_class.astype(i32).reshape(1, N_PAIR), mcls, q,
                     l1_part.reshape(4, 128), cnt_part.reshape(4, 128))
    return (out[0, 0], out[0, 1], out[0, 2])
